# Initial kernel scaffold; baseline (speedup 1.0000x reference)
import jax
import jax.numpy as jnp
from jax.experimental import pallas as pl


def kernel(x, partition_ids, W_emb1, b_emb1, W_emb2, b_emb2, W_g1, b_g1, W_g2, b_g2, W_go, b_go, W_c1, b_c1, ln1_g, ln1_b, W_c2, b_c2, ln2_g, ln2_b, W_c3, b_c3):
    raise NotImplementedError("write your pallas kernel here")


# speedup vs baseline: 17.6998x; 17.6998x over previous
import functools

import jax
import jax.numpy as jnp
from jax.experimental import pallas as pl
from jax.experimental.pallas import tpu as pltpu

P = 1000
A = 8
LANE_F = jnp.float32


def _blockdiag(w, reps):
    return jnp.kron(jnp.eye(reps, dtype=w.dtype), w)


def _passA(x_ref, pid_ref, wb1_ref, bb1_ref, wb2e_ref, bb2e_ref,
           meas_ref, sums1_ref):
    x = x_ref[...]
    x = jnp.where(jnp.isnan(x), 0.0, x)
    x = jnp.where(x == jnp.inf, 1.0, x)
    x = jnp.where(x == -jnp.inf, -1.0, x)
    h = jnp.maximum(
        jax.lax.dot_general(x, wb1_ref[...], (((1,), (0,)), ((), ())),
                            preferred_element_type=LANE_F) + bb1_ref[...],
        0.0)
    meas = jax.lax.dot_general(h, wb2e_ref[...], (((1,), (0,)), ((), ())),
                               preferred_element_type=LANE_F) + bb2e_ref[...]
    meas_ref[...] = meas

    pid = pid_ref[0]
    iot = jax.lax.broadcasted_iota(jnp.int32, (P, pid.shape[1]), 0)
    mask = (iot == pid).astype(LANE_F)
    part = jax.lax.dot_general(mask, meas, (((1,), (0,)), ((), ())),
                               preferred_element_type=LANE_F)

    @pl.when(pl.program_id(0) == 0)
    def _init():
        sums1_ref[...] = jnp.zeros_like(sums1_ref)

    sums1_ref[...] += part


def _passB(meas_ref, pid_ref, t1_ref, wg1_ref, bg1_ref, wg2_ref,
           u_ref, sumsu_ref):
    pid = pid_ref[0]
    B = pid.shape[1]
    iot = jax.lax.broadcasted_iota(jnp.int32, (P, B), 0)
    mask = (iot == pid).astype(LANE_F)

    g1row = jax.lax.dot_general(mask, t1_ref[...], (((0,), (0,)), ((), ())),
                                preferred_element_type=LANE_F)
    cnt = jnp.maximum(g1row[:, 24:25], 1.0)
    pm = meas_ref[...][:, :24] + g1row[:, :24] / cnt
    h1 = jnp.maximum(
        jax.lax.dot_general(pm, wg1_ref[...], (((1,), (0,)), ((), ())),
                            preferred_element_type=LANE_F) + bg1_ref[...],
        0.0)
    wg2 = wg2_ref[...]
    parts = [
        jax.lax.dot_general(h1[:, 64 * a:64 * (a + 1)], wg2,
                            (((1,), (0,)), ((), ())),
                            preferred_element_type=LANE_F)
        for a in range(A)
    ]
    u = jnp.concatenate(parts, axis=1)
    u_ref[...] = u

    part = jax.lax.dot_general(mask, u, (((1,), (0,)), ((), ())),
                               preferred_element_type=LANE_F)

    @pl.when(pl.program_id(0) == 0)
    def _init():
        sumsu_ref[...] = jnp.zeros_like(sumsu_ref)

    sumsu_ref[...] += part


def _passC(u_ref, pid_ref, t1_ref, tu_ref, bg2_ref, wfold_ref, bfold_ref,
           ln1g_ref, ln1b_ref, wc2_ref, bc2_ref, ln2g_ref, ln2b_ref,
           wc3_ref, bc3_ref, out_ref):
    pid = pid_ref[0]
    B = pid.shape[1]
    iot = jax.lax.broadcasted_iota(jnp.int32, (P, B), 0)
    mask = (iot == pid).astype(LANE_F)

    g1row = jax.lax.dot_general(mask, t1_ref[...], (((0,), (0,)), ((), ())),
                                preferred_element_type=LANE_F)
    cnt = jnp.maximum(g1row[:, 24:25], 1.0)
    g2 = jax.lax.dot_general(mask, tu_ref[...], (((0,), (0,)), ((), ())),
                             preferred_element_type=LANE_F)
    h2 = jnp.maximum(u_ref[...] + g2 / cnt + bg2_ref[...], 0.0)

    c = jax.lax.dot_general(h2, wfold_ref[...], (((1,), (0,)), ((), ())),
                            preferred_element_type=LANE_F) + bfold_ref[...]
    mu = jnp.mean(c, axis=-1, keepdims=True)
    var = jnp.mean((c - mu) ** 2, axis=-1, keepdims=True)
    c = (c - mu) * jax.lax.rsqrt(var + 1e-5) * ln1g_ref[...] + ln1b_ref[...]
    c = jnp.maximum(c, 0.0)
    c = jax.lax.dot_general(c, wc2_ref[...], (((1,), (0,)), ((), ())),
                            preferred_element_type=LANE_F) + bc2_ref[...]
    mu = jnp.mean(c, axis=-1, keepdims=True)
    var = jnp.mean((c - mu) ** 2, axis=-1, keepdims=True)
    c = (c - mu) * jax.lax.rsqrt(var + 1e-5) * ln2g_ref[...] + ln2b_ref[...]
    c = jnp.maximum(c, 0.0)
    s = jax.lax.dot_general(c, wc3_ref[...], (((1,), (0,)), ((), ())),
                            preferred_element_type=LANE_F) + bc3_ref[...]
    out_ref[...] = s


def _full(shape):
    return pl.BlockSpec(shape, lambda i: tuple(0 for _ in shape))


def kernel(x, partition_ids, W_emb1, b_emb1, W_emb2, b_emb2, W_g1, b_g1,
           W_g2, b_g2, W_go, b_go, W_c1, b_c1, ln1_g, ln1_b, W_c2, b_c2,
           ln2_g, ln2_b, W_c3, b_c3):
    N = x.shape[0]
    B = 2000 if N % 2000 == 0 else N
    grid = N // B

    f32 = jnp.float32
    x2 = x.reshape(N, A * x.shape[2]).astype(f32)
    pid_row = partition_ids.astype(jnp.int32).reshape(N // B, 1, B)

    Wb1 = _blockdiag(W_emb1, A)
    bb1 = jnp.tile(b_emb1, A).reshape(1, -1)
    Wb2 = _blockdiag(W_emb2, A)
    Wb2e = jnp.concatenate([Wb2, jnp.zeros((Wb2.shape[0], 8), f32)], axis=1)
    bb2e = jnp.concatenate(
        [jnp.tile(b_emb2, A), jnp.ones((1,), f32), jnp.zeros((7,), f32)]
    ).reshape(1, 32)
    Wg1 = _blockdiag(W_g1, A)
    bg1 = jnp.tile(b_g1, A).reshape(1, -1)
    bg2 = jnp.tile(b_g2, A).reshape(1, -1)
    avg = _blockdiag(jnp.ones((3, 1), f32), 1)
    avg = jnp.tile(jnp.eye(3, dtype=f32), (A, 1)) / A
    Wfold = _blockdiag(W_go, A) @ avg @ W_c1
    bfold = (b_go @ W_c1 + b_c1).reshape(1, -1)

    M = B
    meas, sums1 = pl.pallas_call(
        _passA,
        grid=(grid,),
        in_specs=[
            pl.BlockSpec((M, x2.shape[1]), lambda i: (i, 0)),
            pl.BlockSpec((1, 1, M), lambda i: (i, 0, 0)),
            _full(Wb1.shape), _full(bb1.shape),
            _full(Wb2e.shape), _full(bb2e.shape),
        ],
        out_specs=[
            pl.BlockSpec((M, 32), lambda i: (i, 0)),
            pl.BlockSpec((P, 32), lambda i: (0, 0)),
        ],
        out_shape=[
            jax.ShapeDtypeStruct((N, 32), f32),
            jax.ShapeDtypeStruct((P, 32), f32),
        ],
    )(x2, pid_row, Wb1, bb1, Wb2e, bb2e)

    u, sumsu = pl.pallas_call(
        _passB,
        grid=(grid,),
        in_specs=[
            pl.BlockSpec((M, 32), lambda i: (i, 0)),
            pl.BlockSpec((1, 1, M), lambda i: (i, 0, 0)),
            _full((P, 32)),
            _full(Wg1.shape), _full(bg1.shape), _full(W_g2.shape),
        ],
        out_specs=[
            pl.BlockSpec((M, 512), lambda i: (i, 0)),
            pl.BlockSpec((P, 512), lambda i: (0, 0)),
        ],
        out_shape=[
            jax.ShapeDtypeStruct((N, 512), f32),
            jax.ShapeDtypeStruct((P, 512), f32),
        ],
    )(meas, pid_row, sums1, Wg1, bg1, W_g2.astype(f32))

    out = pl.pallas_call(
        _passC,
        grid=(grid,),
        in_specs=[
            pl.BlockSpec((M, 512), lambda i: (i, 0)),
            pl.BlockSpec((1, 1, M), lambda i: (i, 0, 0)),
            _full((P, 32)), _full((P, 512)),
            _full(bg2.shape), _full(Wfold.shape), _full(bfold.shape),
            _full((1, 64)), _full((1, 64)),
            _full(W_c2.shape), _full((1, 32)),
            _full((1, 32)), _full((1, 32)),
            _full(W_c3.shape), _full((1, 1)),
        ],
        out_specs=pl.BlockSpec((M, 1), lambda i: (i, 0)),
        out_shape=jax.ShapeDtypeStruct((N, 1), f32),
    )(u, pid_row, sums1, sumsu, bg2, Wfold, bfold,
      ln1_g.reshape(1, -1), ln1_b.reshape(1, -1),
      W_c2.astype(f32), b_c2.reshape(1, -1),
      ln2_g.reshape(1, -1), ln2_b.reshape(1, -1),
      W_c3.astype(f32), b_c3.reshape(1, 1))

    return out.reshape(N)
